# baseline (speedup 1.0000x reference)
import jax
import jax.numpy as jnp
from jax import lax
from jax.experimental import pallas as pl
from jax.experimental.pallas import tpu as pltpu
from jax.experimental.pallas import tpu_sc as plsc

NUM_CLASSES = 100000
CODE_DIM = 64
K = 4
BATCH = 16384

_info = plsc.get_sparse_core_info()
_NC, _NS, _L = _info.num_cores, _info.num_subcores, _info.num_lanes
_NW = _NC * _NS
_PW = BATCH // _NW
_CH = 128
_NCHUNK = _PW // _CH
_NV = CODE_DIM // _L
_ROWD = K * CODE_DIM
_CP = CODE_DIM + 1


def _sc_body(codes_hbm, pred_hbm, cents_hbm, out_hbm,
             idx_v, slab0, slab1, codep_v, cents0, cents1, out_v,
             spm_v, colbuf_v,
             sem_codes0, sem_codes1, sem_c0, sem_c1):
    wid = lax.axis_index("s") * _NC + lax.axis_index("c")
    base = wid * _PW

    @pl.when(lax.axis_index("s") == 0)
    def _():
        pltpu.sync_copy(cents_hbm.at[pl.ds(0, 256), pl.ds(0, 128)], spm_v)

    plsc.subcore_barrier()

    for c in range(_NCHUNK):
        pltpu.sync_copy(pred_hbm.at[pl.ds(base + c * _CH, _CH)], idx_v.at[c])

    cent_bufs = (cents0, cents1)
    sems = (sem_c0, sem_c1)
    cps = [None, None]
    cps[0] = pltpu.async_copy(cents_hbm.at[idx_v.at[0]], cents0, sem_c0)
    cps[1] = pltpu.async_copy(cents_hbm.at[idx_v.at[1]], cents1, sem_c1)

    iota = lax.iota(jnp.int32, _L)
    lane_last = iota == (_L - 1)

    slab_bufs = (slab0, slab1)
    csems = (sem_codes0, sem_codes1)
    scps = [None, None]
    scps[0] = pltpu.async_copy(codes_hbm.at[:, pl.ds(base, _CH)], slab0,
                               sem_codes0)
    for c in range(_NCHUNK):
        if c + 1 < _NCHUNK:
            nb = (c + 1) % 2
            scps[nb] = pltpu.async_copy(
                codes_hbm.at[:, pl.ds(base + (c + 1) * _CH, _CH)],
                slab_bufs[nb], csems[nb])
        scps[c % 2].wait()
        sbuf = slab_bufs[c % 2]

        def t_group(g, _, c=c, sbuf=sbuf):
            dst0 = (c * _CH + g * _L + iota) * _CP
            for j in range(CODE_DIM):
                vals = sbuf[j, pl.ds(g * _L, _L)]
                plsc.store_scatter(codep_v, [dst0 + j], vals)
            return 0

        lax.fori_loop(0, _CH // _L, t_group, 0)

    for c in range(_NCHUNK):
        cps[c % 2].wait()
        cbuf = cent_bufs[c % 2]

        @plsc.parallel_loop(0, _CH, 1, unroll=4)
        def _item(i, c=c, cbuf=cbuf):
            row = c * _CH + i
            cbase = row * _CP + iota
            code = [plsc.load_gather(codep_v, [cbase + v * _L])
                    for v in range(_NV)]
            s = []
            for k in range(K):
                acc = jnp.abs(code[0] - cbuf[i, pl.ds(k * CODE_DIM, _L)])
                for v in range(1, _NV):
                    t = cbuf[i, pl.ds(k * CODE_DIM + v * _L, _L)]
                    acc = acc + jnp.abs(code[v] - t)
                s.append(plsc.cumsum(acc))
            m = jnp.minimum(jnp.minimum(s[0], s[1]), jnp.minimum(s[2], s[3]))
            m = m * (1.0 / CODE_DIM)
            pltpu.sync_copy(spm_v.at[:, i], colbuf_v)
            m = jnp.minimum(m, colbuf_v[pl.ds(0, _L)] + 1000.0)
            pos = jnp.full((_L,), row, jnp.int32)
            plsc.store_scatter(out_v, [pos], m, mask=lane_last)

        if c + 2 < _NCHUNK:
            nb = c % 2
            cps[nb] = pltpu.async_copy(cents_hbm.at[idx_v.at[c + 2]],
                                       cent_bufs[nb], sems[nb])

    pltpu.sync_copy(out_v, out_hbm.at[pl.ds(base, _PW)])


_mesh = plsc.VectorSubcoreMesh(core_axis_name="c", subcore_axis_name="s")

_sc_kernel = pl.kernel(
    _sc_body,
    mesh=_mesh,
    out_type=jax.ShapeDtypeStruct((BATCH,), jnp.float32),
    scratch_types=[
        pltpu.VMEM((_NCHUNK, _CH), jnp.int32),
        pltpu.VMEM((CODE_DIM, _CH), jnp.float32),
        pltpu.VMEM((CODE_DIM, _CH), jnp.float32),
        pltpu.VMEM((_PW * _CP,), jnp.float32),
        pltpu.VMEM((_CH, _ROWD), jnp.float32),
        pltpu.VMEM((_CH, _ROWD), jnp.float32),
        pltpu.VMEM((_PW,), jnp.float32),
        pltpu.VMEM_SHARED((256, 128), jnp.float32),
        pltpu.VMEM((256,), jnp.float32),
        pltpu.SemaphoreType.DMA,
        pltpu.SemaphoreType.DMA,
        pltpu.SemaphoreType.DMA,
        pltpu.SemaphoreType.DMA,
    ],
    compiler_params=pltpu.CompilerParams(needs_layout_passes=False),
)


def kernel(codes, pred_class, centroids):
    pred = pred_class.astype(jnp.int32)
    cents = centroids.reshape(NUM_CLASSES, _ROWD)
    return _sc_kernel(codes.T, pred, cents)


# speedup vs baseline: 1.3751x; 1.3751x over previous
import functools

import jax
import jax.numpy as jnp
from jax import lax
from jax.experimental import pallas as pl
from jax.experimental.pallas import tpu as pltpu
from jax.experimental.pallas import tpu_sc as plsc

NUM_CLASSES = 100000
CODE_DIM = 64
K = 4
BATCH = 16384

_info = plsc.get_sparse_core_info()
_NC, _NS, _L = _info.num_cores, _info.num_subcores, _info.num_lanes
_NW = _NC * _NS
_PW = BATCH // _NW
_CH = 128
_NCHUNK = _PW // _CH
_NV = CODE_DIM // _L
_ROWD = K * CODE_DIM
_CP = CODE_DIM + 1
_NQ = 4
_QC = NUM_CLASSES // _NQ
_TRASH = _PW


def _sc_body(codes_hbm, pred_hbm, cents_hbm, out_hbm,
             pred_v, ids_v, idx_v, slab0, slab1, codep_v,
             cents0, cents1, out_v,
             sem_codes0, sem_codes1, sem_c0, sem_c1,
             *, lo):
    wid = lax.axis_index("s") * _NC + lax.axis_index("c")
    base = wid * _PW

    pltpu.sync_copy(pred_hbm.at[pl.ds(base, _PW)], pred_v)

    iota = lax.iota(jnp.int32, _L)
    lane_last = iota == (_L - 1)

    big = jnp.full((_L,), 1e30, jnp.float32)
    trash = jnp.full((_L,), _TRASH, jnp.int32)
    for g in range(_PW // _L):
        idx_v[pl.ds(g * _L, _L)] = base + g * _L + iota
        ids_v[pl.ds(g * _L, _L)] = trash
        out_v[pl.ds(g * _L, _L)] = big
    out_v[pl.ds(_PW, _L)] = big
    ids_v[pl.ds(_PW, _L)] = trash

    def scan_g(g, cnt):
        c16 = pred_v[pl.ds(g * _L, _L)]
        inr = (c16 >= lo) & (c16 < lo + _QC)
        mi = inr.astype(jnp.int32)
        pos = plsc.cumsum(mi) - 1 + cnt
        plsc.store_scatter(ids_v, [pos], g * _L + iota, mask=inr)
        plsc.store_scatter(idx_v, [pos], c16 - lo, mask=inr)
        return cnt + plsc.all_reduce_population_count(inr)

    cnt_vec = lax.fori_loop(0, _PW // _L, scan_g,
                            jnp.zeros((_L,), jnp.int32))
    count = jnp.max(cnt_vec)

    cent_bufs = (cents0, cents1)
    sems = (sem_c0, sem_c1)
    cps = [None, None]
    cps[0] = pltpu.async_copy(cents_hbm.at[idx_v.at[pl.ds(0, _CH)]],
                              cents0, sem_c0)
    cps[1] = pltpu.async_copy(cents_hbm.at[idx_v.at[pl.ds(_CH, _CH)]],
                              cents1, sem_c1)

    slab_bufs = (slab0, slab1)
    csems = (sem_codes0, sem_codes1)
    scps = [None, None]
    scps[0] = pltpu.async_copy(codes_hbm.at[:, pl.ds(base, _CH)], slab0,
                               sem_codes0)
    for c in range(_NCHUNK):
        if c + 1 < _NCHUNK:
            nb = (c + 1) % 2
            scps[nb] = pltpu.async_copy(
                codes_hbm.at[:, pl.ds(base + (c + 1) * _CH, _CH)],
                slab_bufs[nb], csems[nb])
        scps[c % 2].wait()
        sbuf = slab_bufs[c % 2]

        def t_group(g, _, c=c, sbuf=sbuf):
            dst0 = (c * _CH + g * _L + iota) * _CP
            for j in range(CODE_DIM):
                vals = sbuf[j, pl.ds(g * _L, _L)]
                plsc.store_scatter(codep_v, [dst0 + j], vals)
            return 0

        lax.fori_loop(0, _CH // _L, t_group, 0)

    for c in range(_NCHUNK):
        cps[c % 2].wait()
        cbuf = cent_bufs[c % 2]

        nd = jnp.clip(count - c * _CH, 0, _CH)
        ngroups = (nd + _L - 1) // _L

        def grp(g, _, c=c, cbuf=cbuf):
            ids16 = ids_v[pl.ds(c * _CH + g * _L, _L)]
            for u in range(_L):
                ld = g * _L + u
                p = ids16[u]
                cbase = p * _CP + iota
                code = [plsc.load_gather(codep_v, [cbase + v * _L])
                        for v in range(_NV)]
                s = []
                for k in range(K):
                    acc = jnp.abs(code[0] - cbuf[ld, pl.ds(k * CODE_DIM, _L)])
                    for v in range(1, _NV):
                        t = cbuf[ld, pl.ds(k * CODE_DIM + v * _L, _L)]
                        acc = acc + jnp.abs(code[v] - t)
                    s.append(plsc.cumsum(acc))
                m = jnp.minimum(jnp.minimum(s[0], s[1]),
                                jnp.minimum(s[2], s[3]))
                m = m * (1.0 / CODE_DIM)
                pos = jnp.full((_L,), p, jnp.int32)
                plsc.store_scatter(out_v, [pos], m, mask=lane_last)
            return 0

        lax.fori_loop(0, ngroups, grp, 0)

        if c + 2 < _NCHUNK:
            nb = c % 2
            cps[nb] = pltpu.async_copy(
                cents_hbm.at[idx_v.at[pl.ds((c + 2) * _CH, _CH)]],
                cent_bufs[nb], sems[nb])

    pltpu.sync_copy(out_v.at[pl.ds(0, _PW)], out_hbm.at[pl.ds(base, _PW)])


_mesh = plsc.VectorSubcoreMesh(core_axis_name="c", subcore_axis_name="s")

_scratch = [
    pltpu.VMEM((_PW,), jnp.int32),
    pltpu.VMEM((_PW + _L,), jnp.int32),
    pltpu.VMEM((_PW,), jnp.int32),
    pltpu.VMEM((CODE_DIM, _CH), jnp.float32),
    pltpu.VMEM((CODE_DIM, _CH), jnp.float32),
    pltpu.VMEM(((_PW + 1) * _CP + _L,), jnp.float32),
    pltpu.VMEM((_CH, _ROWD), jnp.float32),
    pltpu.VMEM((_CH, _ROWD), jnp.float32),
    pltpu.VMEM((_PW + _L,), jnp.float32),
    pltpu.SemaphoreType.DMA,
    pltpu.SemaphoreType.DMA,
    pltpu.SemaphoreType.DMA,
    pltpu.SemaphoreType.DMA,
]

_sc_kernels = [
    pl.kernel(
        functools.partial(_sc_body, lo=q * _QC),
        mesh=_mesh,
        out_type=jax.ShapeDtypeStruct((BATCH,), jnp.float32),
        scratch_types=_scratch,
        compiler_params=pltpu.CompilerParams(needs_layout_passes=False),
        name=f"codebook_q{q}",
    )
    for q in range(_NQ)
]


def kernel(codes, pred_class, centroids):
    pred = pred_class.astype(jnp.int32)
    cents = centroids.reshape(NUM_CLASSES, _ROWD)
    codes_t = codes.T
    outs = [
        _sc_kernels[q](codes_t, pred,
                       lax.slice_in_dim(cents, q * _QC, (q + 1) * _QC, axis=0))
        for q in range(_NQ)
    ]
    return functools.reduce(jnp.minimum, outs)


# speedup vs baseline: 2.4959x; 1.8150x over previous
import jax
import jax.numpy as jnp
from jax import lax
from jax.experimental import pallas as pl
from jax.experimental.pallas import tpu as pltpu
from jax.experimental.pallas import tpu_sc as plsc

NUM_CLASSES = 100000
CODE_DIM = 64
K = 4
BATCH = 16384

_info = plsc.get_sparse_core_info()
_NC, _NS, _L = _info.num_cores, _info.num_subcores, _info.num_lanes
_NW = _NC * _NS
_PW = BATCH // _NW
_CH = 128
_NCHUNK = _PW // _CH
_NV = CODE_DIM // _L
_ROWD = K * CODE_DIM
_CP = CODE_DIM + 1


def _sc_body(codes_hbm, pred_hbm, cents_hbm, out_hbm,
             idx_v, slab0, slab1, codep_v, cents0, cents1, out_v,
             sem_codes0, sem_codes1, sem_c0, sem_c1):
    wid = lax.axis_index("s") * _NC + lax.axis_index("c")
    base = wid * _PW

    for c in range(_NCHUNK):
        pltpu.sync_copy(pred_hbm.at[pl.ds(base + c * _CH, _CH)], idx_v.at[c])

    cent_bufs = (cents0, cents1)
    sems = (sem_c0, sem_c1)
    cps = [None, None]
    cps[0] = pltpu.async_copy(cents_hbm.at[idx_v.at[0]], cents0, sem_c0)
    cps[1] = pltpu.async_copy(cents_hbm.at[idx_v.at[1]], cents1, sem_c1)

    iota = lax.iota(jnp.int32, _L)
    lane_last = iota == (_L - 1)

    slab_bufs = (slab0, slab1)
    csems = (sem_codes0, sem_codes1)
    scps = [None, None]
    scps[0] = pltpu.async_copy(codes_hbm.at[:, pl.ds(base, _CH)], slab0,
                               sem_codes0)
    for c in range(_NCHUNK):
        if c + 1 < _NCHUNK:
            nb = (c + 1) % 2
            scps[nb] = pltpu.async_copy(
                codes_hbm.at[:, pl.ds(base + (c + 1) * _CH, _CH)],
                slab_bufs[nb], csems[nb])
        scps[c % 2].wait()
        sbuf = slab_bufs[c % 2]

        def t_group(g, _, c=c, sbuf=sbuf):
            dst0 = (c * _CH + g * _L + iota) * _CP
            for j in range(CODE_DIM):
                vals = sbuf[j, pl.ds(g * _L, _L)]
                plsc.store_scatter(codep_v, [dst0 + j], vals)
            return 0

        lax.fori_loop(0, _CH // _L, t_group, 0)

    for c in range(_NCHUNK):
        cps[c % 2].wait()
        cbuf = cent_bufs[c % 2]

        @plsc.parallel_loop(0, _CH, 1, unroll=8)
        def _item(i, c=c, cbuf=cbuf):
            row = c * _CH + i
            cbase = row * _CP + iota
            code = [plsc.load_gather(codep_v, [cbase + v * _L])
                    for v in range(_NV)]
            s = []
            for k in range(K):
                acc = jnp.abs(code[0] - cbuf[i, pl.ds(k * CODE_DIM, _L)])
                for v in range(1, _NV):
                    t = cbuf[i, pl.ds(k * CODE_DIM + v * _L, _L)]
                    acc = acc + jnp.abs(code[v] - t)
                s.append(plsc.cumsum(acc))
            m = jnp.minimum(jnp.minimum(s[0], s[1]), jnp.minimum(s[2], s[3]))
            m = m * (1.0 / CODE_DIM)
            pos = jnp.full((_L,), row, jnp.int32)
            plsc.store_scatter(out_v, [pos], m, mask=lane_last)

        if c + 2 < _NCHUNK:
            nb = c % 2
            cps[nb] = pltpu.async_copy(cents_hbm.at[idx_v.at[c + 2]],
                                       cent_bufs[nb], sems[nb])

    pltpu.sync_copy(out_v, out_hbm.at[pl.ds(base, _PW)])


_mesh = plsc.VectorSubcoreMesh(core_axis_name="c", subcore_axis_name="s")

_sc_kernel = pl.kernel(
    _sc_body,
    mesh=_mesh,
    out_type=jax.ShapeDtypeStruct((BATCH,), jnp.float32),
    scratch_types=[
        pltpu.VMEM((_NCHUNK, _CH), jnp.int32),
        pltpu.VMEM((CODE_DIM, _CH), jnp.float32),
        pltpu.VMEM((CODE_DIM, _CH), jnp.float32),
        pltpu.VMEM((_PW * _CP,), jnp.float32),
        pltpu.VMEM((_CH, _ROWD), jnp.float32),
        pltpu.VMEM((_CH, _ROWD), jnp.float32),
        pltpu.VMEM((_PW,), jnp.float32),
        pltpu.SemaphoreType.DMA,
        pltpu.SemaphoreType.DMA,
        pltpu.SemaphoreType.DMA,
        pltpu.SemaphoreType.DMA,
    ],
    compiler_params=pltpu.CompilerParams(needs_layout_passes=False),
)


def kernel(codes, pred_class, centroids):
    pred = pred_class.astype(jnp.int32)
    cents = centroids.reshape(NUM_CLASSES, _ROWD)
    return _sc_kernel(codes.T, pred, cents)


# speedup vs baseline: 2.5796x; 1.0335x over previous
import jax
import jax.numpy as jnp
from jax import lax
from jax.experimental import pallas as pl
from jax.experimental.pallas import tpu as pltpu
from jax.experimental.pallas import tpu_sc as plsc

NUM_CLASSES = 100000
CODE_DIM = 64
K = 4
BATCH = 16384

_info = plsc.get_sparse_core_info()
_NC, _NS, _L = _info.num_cores, _info.num_subcores, _info.num_lanes
_NW = _NC * _NS
_PW = BATCH // _NW
_CH = 128
_NCHUNK = _PW // _CH
_NV = CODE_DIM // _L
_ROWD = K * CODE_DIM
_CP = CODE_DIM + 1


def _sc_body(codes_hbm, pred_hbm, cents_hbm, out_hbm,
             idx_v, slab0, slab1, codep_v, cents0, cents1, out_v,
             sem_codes0, sem_codes1, sem_c0, sem_c1):
    wid = lax.axis_index("s") * _NC + lax.axis_index("c")
    base = wid * _PW

    pltpu.sync_copy(pred_hbm.at[pl.ds(base, _PW)], idx_v)

    cent_bufs = (cents0, cents1)
    sems = (sem_c0, sem_c1)
    cps = [None, None]
    cps[0] = pltpu.async_copy(cents_hbm.at[idx_v.at[pl.ds(0, _CH)]],
                              cents0, sem_c0)
    cps[1] = pltpu.async_copy(cents_hbm.at[idx_v.at[pl.ds(_CH, _CH)]],
                              cents1, sem_c1)

    iota = lax.iota(jnp.int32, _L)
    lane_last = iota == (_L - 1)

    slab_bufs = (slab0, slab1)
    csems = (sem_codes0, sem_codes1)
    scps = [None, None]
    scps[0] = pltpu.async_copy(codes_hbm.at[:, pl.ds(base, _CH)], slab0,
                               sem_codes0)
    for c in range(_NCHUNK):
        if c + 1 < _NCHUNK:
            nb = (c + 1) % 2
            scps[nb] = pltpu.async_copy(
                codes_hbm.at[:, pl.ds(base + (c + 1) * _CH, _CH)],
                slab_bufs[nb], csems[nb])
        scps[c % 2].wait()
        sbuf = slab_bufs[c % 2]

        def t_group(g, _, c=c, sbuf=sbuf):
            dst0 = (c * _CH + g * _L + iota) * _CP
            for j in range(CODE_DIM):
                vals = sbuf[j, pl.ds(g * _L, _L)]
                plsc.store_scatter(codep_v, [dst0 + j], vals)
            return 0

        lax.fori_loop(0, _CH // _L, t_group, 0)

    for c in range(_NCHUNK):
        cps[c % 2].wait()
        cbuf = cent_bufs[c % 2]

        @plsc.parallel_loop(0, _CH, 1, unroll=4)
        def _item(i, c=c, cbuf=cbuf):
            row = c * _CH + i
            cbase = row * _CP + iota
            code = [plsc.load_gather(codep_v, [cbase + v * _L])
                    for v in range(_NV)]
            s = []
            for k in range(K):
                acc = jnp.abs(code[0] - cbuf[i, pl.ds(k * CODE_DIM, _L)])
                for v in range(1, _NV):
                    t = cbuf[i, pl.ds(k * CODE_DIM + v * _L, _L)]
                    acc = acc + jnp.abs(code[v] - t)
                s.append(plsc.cumsum(acc))
            m = jnp.minimum(jnp.minimum(s[0], s[1]), jnp.minimum(s[2], s[3]))
            m = m * (1.0 / CODE_DIM)
            pos = jnp.full((_L,), row, jnp.int32)
            plsc.store_scatter(out_v, [pos], m, mask=lane_last)

        if c + 2 < _NCHUNK:
            nb = c % 2
            cps[nb] = pltpu.async_copy(
                cents_hbm.at[idx_v.at[pl.ds((c + 2) * _CH, _CH)]],
                cent_bufs[nb], sems[nb])

    pltpu.sync_copy(out_v, out_hbm.at[pl.ds(base, _PW)])


_mesh = plsc.VectorSubcoreMesh(core_axis_name="c", subcore_axis_name="s")

_sc_kernel = pl.kernel(
    _sc_body,
    mesh=_mesh,
    out_type=jax.ShapeDtypeStruct((BATCH,), jnp.float32),
    scratch_types=[
        pltpu.VMEM((_PW,), jnp.int32),
        pltpu.VMEM((CODE_DIM, _CH), jnp.float32),
        pltpu.VMEM((CODE_DIM, _CH), jnp.float32),
        pltpu.VMEM((_PW * _CP,), jnp.float32),
        pltpu.VMEM((_CH, _ROWD), jnp.float32),
        pltpu.VMEM((_CH, _ROWD), jnp.float32),
        pltpu.VMEM((_PW,), jnp.float32),
        pltpu.SemaphoreType.DMA,
        pltpu.SemaphoreType.DMA,
        pltpu.SemaphoreType.DMA,
        pltpu.SemaphoreType.DMA,
    ],
    compiler_params=pltpu.CompilerParams(needs_layout_passes=False),
)


def kernel(codes, pred_class, centroids):
    pred = pred_class.astype(jnp.int32)
    cents = centroids.reshape(NUM_CLASSES, _ROWD)
    return _sc_kernel(codes.T, pred, cents)
